# Initial kernel scaffold; baseline (speedup 1.0000x reference)
import functools

import jax
import jax.numpy as jnp
from jax import lax
from jax.experimental import pallas as pl
from jax.experimental.pallas import tpu as pltpu
from jax.experimental.pallas import tpu_sc as plsc

NC = 2
NS = 16
CH = 128
BN = 2000


def _sc_mesh():
    return plsc.VectorSubcoreMesh(
        core_axis_name="c", subcore_axis_name="s",
        num_cores=NC, num_subcores=NS)


def _deg_partials(dstp, ones, zeros, NP, cpw):
    rps = NP // NS

    @functools.partial(
        pl.kernel,
        out_type=jax.ShapeDtypeStruct((NC, NP, 16), jnp.float32),
        mesh=_sc_mesh(),
        scratch_types=[
            pltpu.VMEM_SHARED((NP, 16), jnp.float32),
            pltpu.VMEM((CH,), jnp.int32),
            pltpu.VMEM((CH, 16), jnp.float32),
        ],
    )
    def run(dst_hbm, ones_hbm, zeros_hbm, out_hbm, acc, didx, ones_v):
        c = lax.axis_index("c")
        s = lax.axis_index("s")
        wid = s * NC + c
        base = wid * (cpw * CH)
        pltpu.sync_copy(zeros_hbm, acc.at[pl.ds(s * rps, rps)])
        pltpu.sync_copy(ones_hbm, ones_v)
        plsc.subcore_barrier()

        @pl.loop(0, cpw)
        def _(g):
            pltpu.sync_copy(dst_hbm.at[pl.ds(base + g * CH, CH)], didx)
            pltpu.sync_copy(ones_v, acc.at[didx], add=True)

        plsc.subcore_barrier()
        pltpu.sync_copy(acc.at[pl.ds(s * rps, rps)],
                        out_hbm.at[c, pl.ds(s * rps, rps)])

    return run(dstp, ones, zeros)


def _edge_pass(hn, srcp, dstp, zeros, NP, cpw):
    n, h = hn.shape
    rps = NP // NS

    @functools.partial(
        pl.kernel,
        out_type=jax.ShapeDtypeStruct((NC, NP, h), jnp.float32),
        mesh=_sc_mesh(),
        scratch_types=[
            pltpu.VMEM_SHARED((NP, h), jnp.float32),
            pltpu.VMEM((CH,), jnp.int32),
            pltpu.VMEM((CH,), jnp.int32),
            pltpu.VMEM((CH, h), jnp.float32),
            pltpu.SemaphoreType.DMA,
        ],
    )
    def run(hn_hbm, src_hbm, dst_hbm, zeros_hbm, out_hbm,
            acc, sidx, didx, rows, sem):
        c = lax.axis_index("c")
        s = lax.axis_index("s")
        wid = s * NC + c
        base = wid * (cpw * CH)
        pltpu.sync_copy(zeros_hbm, acc.at[pl.ds(s * rps, rps)])
        plsc.subcore_barrier()

        @pl.loop(0, cpw)
        def _(g):
            e0 = base + g * CH
            pltpu.sync_copy(src_hbm.at[pl.ds(e0, CH)], sidx)
            pltpu.sync_copy(dst_hbm.at[pl.ds(e0, CH)], didx)
            pltpu.async_copy(hn_hbm.at[sidx], rows, sem).wait()
            pltpu.sync_copy(rows, acc.at[didx], add=True)

        plsc.subcore_barrier()
        pltpu.sync_copy(acc.at[pl.ds(s * rps, rps)],
                        out_hbm.at[c, pl.ds(s * rps, rps)])

    return run(hn, srcp, dstp, zeros)


def _norm_from(dp):
    deg = dp[0, :, 0] + dp[1, :, 0]
    return lax.rsqrt(jnp.maximum(deg, 1.0))


def _mm(a, w):
    return jnp.dot(a, w, precision=lax.Precision.HIGHEST,
                   preferred_element_type=jnp.float32)


def _stage_a(degp, x):
    n, d = x.shape

    def body(dp_ref, x_ref, out_ref):
        norm = _norm_from(dp_ref[...])
        out_ref[...] = x_ref[...] * norm[:, None]

    return pl.pallas_call(
        body,
        grid=(n // BN,),
        in_specs=[
            pl.BlockSpec((2, BN, 16), lambda i: (0, i, 0)),
            pl.BlockSpec((BN, d), lambda i: (i, 0)),
        ],
        out_specs=pl.BlockSpec((BN, d), lambda i: (i, 0)),
        out_shape=jax.ShapeDtypeStruct((n, d), jnp.float32),
    )(degp, x)


def _stage_b(degp, aggp, W, b, n):
    d, h = W.shape

    def body(dp_ref, ap_ref, w_ref, b_ref, out_ref):
        norm = _norm_from(dp_ref[...])
        ap = ap_ref[...]
        agg = (ap[0] + ap[1]) * norm[:, None]
        hh = jnp.maximum(_mm(agg, w_ref[...]) + b_ref[...], 0.0)
        out_ref[...] = hh * norm[:, None]

    return pl.pallas_call(
        body,
        grid=(n // BN,),
        in_specs=[
            pl.BlockSpec((2, BN, 16), lambda i: (0, i, 0)),
            pl.BlockSpec((2, BN, d), lambda i: (0, i, 0)),
            pl.BlockSpec((d, h), lambda i: (0, 0)),
            pl.BlockSpec((1, h), lambda i: (0, 0)),
        ],
        out_specs=pl.BlockSpec((BN, h), lambda i: (i, 0)),
        out_shape=jax.ShapeDtypeStruct((n, h), jnp.float32),
    )(degp, aggp, W, b)


def _stage_c(degp, aggp, W2, b2, Wl1, bl1, Wl2, bl2, n):
    d, h = W2.shape
    h1 = Wl1.shape[1]
    c = Wl2.shape[1]

    def body(dp_ref, ap_ref, w2_ref, b2_ref, wl1_ref, bl1_ref,
             wl2_ref, bl2_ref, out_ref):
        norm = _norm_from(dp_ref[...])
        ap = ap_ref[...]
        agg = (ap[0] + ap[1]) * norm[:, None]
        hh = jnp.maximum(_mm(agg, w2_ref[...]) + b2_ref[...], 0.0)
        hh = jnp.maximum(_mm(hh, wl1_ref[...]) + bl1_ref[...], 0.0)
        out_ref[...] = _mm(hh, wl2_ref[...]) + bl2_ref[...]

    return pl.pallas_call(
        body,
        grid=(n // BN,),
        in_specs=[
            pl.BlockSpec((2, BN, 16), lambda i: (0, i, 0)),
            pl.BlockSpec((2, BN, d), lambda i: (0, i, 0)),
            pl.BlockSpec((d, h), lambda i: (0, 0)),
            pl.BlockSpec((1, h), lambda i: (0, 0)),
            pl.BlockSpec((h, h1), lambda i: (0, 0)),
            pl.BlockSpec((1, h1), lambda i: (0, 0)),
            pl.BlockSpec((h1, c), lambda i: (0, 0)),
            pl.BlockSpec((1, c), lambda i: (0, 0)),
        ],
        out_specs=pl.BlockSpec((BN, c), lambda i: (i, 0)),
        out_shape=jax.ShapeDtypeStruct((n, c), jnp.float32),
    )(degp, aggp, W2, b2, Wl1, bl1, Wl2, bl2)


def kernel(x, edge_index, W1, b1, W2, b2, Wl1, bl1, Wl2, bl2):
    n, d = x.shape
    h = W1.shape[1]
    e = edge_index.shape[1]

    cpw = -(-e // (NC * NS * CH))
    ep = NC * NS * CH * cpw
    np_rows = ((n // NS) + 3) * NS

    src = edge_index[0]
    dst = edge_index[1]
    pad = ep - e
    if pad:
        pi = lax.iota(jnp.int32, pad)
        src = jnp.concatenate([src, pi % n])
        dst = jnp.concatenate([dst, n + pi % (np_rows - n)])

    zeros_h = jnp.zeros((np_rows // NS, h), jnp.float32)
    zeros16 = jnp.zeros((np_rows // NS, 16), jnp.float32)
    ones16 = jnp.ones((CH, 16), jnp.float32)

    degp = _deg_partials(dst, ones16, zeros16, np_rows, cpw)
    hn0 = _stage_a(degp, x)
    agg1 = _edge_pass(hn0, src, dst, zeros_h, np_rows, cpw)
    hn1 = _stage_b(degp, agg1, W1, b1.reshape(1, -1), n)
    agg2 = _edge_pass(hn1, src, dst, zeros_h, np_rows, cpw)
    return _stage_c(degp, agg2, W2, b2.reshape(1, -1),
                    Wl1, bl1.reshape(1, -1), Wl2, bl2.reshape(1, -1), n)


# speedup vs baseline: 5.4973x; 5.4973x over previous
import functools

import jax
import jax.numpy as jnp
from jax import lax
from jax.experimental import pallas as pl
from jax.experimental.pallas import tpu as pltpu
from jax.experimental.pallas import tpu_sc as plsc

NC = 2
NS = 16
CH = 128
BN = 2000


def _sc_mesh():
    return plsc.VectorSubcoreMesh(
        core_axis_name="c", subcore_axis_name="s",
        num_cores=NC, num_subcores=NS)


def _deg_partials(dstp, ones, zeros, NP, cpw):
    w = ones.shape[1]
    rps = NP // NS

    @functools.partial(
        pl.kernel,
        out_type=jax.ShapeDtypeStruct((NC, NP, w), jnp.float32),
        mesh=_sc_mesh(),
        scratch_types=[
            pltpu.VMEM_SHARED((NP, w), jnp.float32),
            pltpu.VMEM((CH,), jnp.int32),
            pltpu.VMEM((CH, w), jnp.float32),
        ],
    )
    def run(dst_hbm, ones_hbm, zeros_hbm, out_hbm, acc, didx, ones_v):
        c = lax.axis_index("c")
        s = lax.axis_index("s")
        wid = s * NC + c
        base = wid * (cpw * CH)
        pltpu.sync_copy(zeros_hbm, acc.at[pl.ds(s * rps, rps)])
        pltpu.sync_copy(ones_hbm, ones_v)
        plsc.subcore_barrier()

        @pl.loop(0, cpw)
        def _(g):
            pltpu.sync_copy(dst_hbm.at[pl.ds(base + g * CH, CH)], didx)
            pltpu.sync_copy(ones_v, acc.at[didx], add=True)

        plsc.subcore_barrier()
        pltpu.sync_copy(acc.at[pl.ds(s * rps, rps)],
                        out_hbm.at[c, pl.ds(s * rps, rps)])

    return run(dstp, ones, zeros)


def _edge_pass(hn, srcp, dstp, zeros, NP, cpw):
    n, h = hn.shape
    rps = NP // NS

    @functools.partial(
        pl.kernel,
        out_type=jax.ShapeDtypeStruct((NC, NP, h), jnp.float32),
        mesh=_sc_mesh(),
        scratch_types=[
            pltpu.VMEM_SHARED((NP, h), jnp.float32),
            pltpu.VMEM((CH,), jnp.int32),
            pltpu.VMEM((CH,), jnp.int32),
            pltpu.VMEM((CH, h), jnp.float32),
            pltpu.SemaphoreType.DMA,
        ],
    )
    def run(hn_hbm, src_hbm, dst_hbm, zeros_hbm, out_hbm,
            acc, sidx, didx, rows, sem):
        c = lax.axis_index("c")
        s = lax.axis_index("s")
        wid = s * NC + c
        base = wid * (cpw * CH)
        pltpu.sync_copy(zeros_hbm, acc.at[pl.ds(s * rps, rps)])
        plsc.subcore_barrier()

        @pl.loop(0, cpw)
        def _(g):
            e0 = base + g * CH
            pltpu.sync_copy(src_hbm.at[pl.ds(e0, CH)], sidx)
            pltpu.sync_copy(dst_hbm.at[pl.ds(e0, CH)], didx)
            pltpu.async_copy(hn_hbm.at[sidx], rows, sem).wait()
            pltpu.sync_copy(rows, acc.at[didx], add=True)

        plsc.subcore_barrier()
        pltpu.sync_copy(acc.at[pl.ds(s * rps, rps)],
                        out_hbm.at[c, pl.ds(s * rps, rps)])

    return run(hn, srcp, dstp, zeros)


def _norm_from(dp):
    deg = dp[0, :, 0] + dp[1, :, 0]
    return lax.rsqrt(jnp.maximum(deg, 1.0))


def _mm(a, w):
    return jnp.dot(a, w, precision=lax.Precision.HIGHEST,
                   preferred_element_type=jnp.float32)


def _stage_a(degp, x):
    n, d = x.shape

    def body(dp_ref, x_ref, out_ref):
        norm = _norm_from(dp_ref[...])
        out_ref[...] = x_ref[...] * norm[:, None]

    return pl.pallas_call(
        body,
        grid=(n // BN,),
        in_specs=[
            pl.BlockSpec((2, BN, 128), lambda i: (0, i, 0)),
            pl.BlockSpec((BN, d), lambda i: (i, 0)),
        ],
        out_specs=pl.BlockSpec((BN, d), lambda i: (i, 0)),
        out_shape=jax.ShapeDtypeStruct((n, d), jnp.float32),
    )(degp, x)


def _stage_b(degp, aggp, W, b, n):
    d, h = W.shape

    def body(dp_ref, ap_ref, w_ref, b_ref, out_ref):
        norm = _norm_from(dp_ref[...])
        ap = ap_ref[...]
        agg = (ap[0] + ap[1]) * norm[:, None]
        hh = jnp.maximum(_mm(agg, w_ref[...]) + b_ref[...], 0.0)
        out_ref[...] = hh * norm[:, None]

    return pl.pallas_call(
        body,
        grid=(n // BN,),
        in_specs=[
            pl.BlockSpec((2, BN, 128), lambda i: (0, i, 0)),
            pl.BlockSpec((2, BN, d), lambda i: (0, i, 0)),
            pl.BlockSpec((d, h), lambda i: (0, 0)),
            pl.BlockSpec((1, h), lambda i: (0, 0)),
        ],
        out_specs=pl.BlockSpec((BN, h), lambda i: (i, 0)),
        out_shape=jax.ShapeDtypeStruct((n, h), jnp.float32),
    )(degp, aggp, W, b)


def _stage_c(degp, aggp, W2, b2, Wl1, bl1, Wl2, bl2, n):
    d, h = W2.shape
    h1 = Wl1.shape[1]
    c = Wl2.shape[1]

    def body(dp_ref, ap_ref, w2_ref, b2_ref, wl1_ref, bl1_ref,
             wl2_ref, bl2_ref, out_ref):
        norm = _norm_from(dp_ref[...])
        ap = ap_ref[...]
        agg = (ap[0] + ap[1]) * norm[:, None]
        hh = jnp.maximum(_mm(agg, w2_ref[...]) + b2_ref[...], 0.0)
        hh = jnp.maximum(_mm(hh, wl1_ref[...]) + bl1_ref[...], 0.0)
        out_ref[...] = _mm(hh, wl2_ref[...]) + bl2_ref[...]

    return pl.pallas_call(
        body,
        grid=(n // BN,),
        in_specs=[
            pl.BlockSpec((2, BN, 128), lambda i: (0, i, 0)),
            pl.BlockSpec((2, BN, d), lambda i: (0, i, 0)),
            pl.BlockSpec((d, h), lambda i: (0, 0)),
            pl.BlockSpec((1, h), lambda i: (0, 0)),
            pl.BlockSpec((h, h1), lambda i: (0, 0)),
            pl.BlockSpec((1, h1), lambda i: (0, 0)),
            pl.BlockSpec((h1, c), lambda i: (0, 0)),
            pl.BlockSpec((1, c), lambda i: (0, 0)),
        ],
        out_specs=pl.BlockSpec((BN, c), lambda i: (i, 0)),
        out_shape=jax.ShapeDtypeStruct((n, c), jnp.float32),
    )(degp, aggp, W2, b2, Wl1, bl1, Wl2, bl2)


def kernel(x, edge_index, W1, b1, W2, b2, Wl1, bl1, Wl2, bl2):
    n, d = x.shape
    h = W1.shape[1]
    e = edge_index.shape[1]

    cpw = -(-e // (NC * NS * CH))
    ep = NC * NS * CH * cpw
    np_rows = -(-(n + 1) // (NS * 8)) * (NS * 8)

    src = edge_index[0]
    dst = edge_index[1]
    pad = ep - e
    if pad:
        pi = lax.iota(jnp.int32, pad)
        src = jnp.concatenate([src, pi % n])
        dst = jnp.concatenate([dst, n + pi % (np_rows - n)])

    zeros_h = jnp.zeros((np_rows // NS, h), jnp.float32)
    ones_h = jnp.ones((CH, h), jnp.float32)

    degp = _deg_partials(dst, ones_h, zeros_h, np_rows, cpw)
    hn0 = _stage_a(degp, x)
    agg1 = _edge_pass(hn0, src, dst, zeros_h, np_rows, cpw)
    hn1 = _stage_b(degp, agg1, W1, b1.reshape(1, -1), n)
    agg2 = _edge_pass(hn1, src, dst, zeros_h, np_rows, cpw)
    return _stage_c(degp, agg2, W2, b2.reshape(1, -1),
                    Wl1, bl1.reshape(1, -1), Wl2, bl2.reshape(1, -1), n)


# speedup vs baseline: 8.2422x; 1.4993x over previous
import functools

import jax
import jax.numpy as jnp
from jax import lax
from jax.experimental import pallas as pl
from jax.experimental.pallas import tpu as pltpu
from jax.experimental.pallas import tpu_sc as plsc

NC = 2
DW = 128
NS = 16
CH = 128
BN = 2000


def _sc_mesh():
    return plsc.VectorSubcoreMesh(
        core_axis_name="c", subcore_axis_name="s",
        num_cores=NC, num_subcores=NS)


NB = 4


def _deg_partials(dst3, ones, zeros, NP, cpw):
    w = ones.shape[1]
    rps = NP // NS

    @functools.partial(
        pl.kernel,
        out_type=jax.ShapeDtypeStruct((NC, NP, w), jnp.float32),
        mesh=_sc_mesh(),
        scratch_types=[
            pltpu.VMEM_SHARED((NP, w), jnp.float32),
            pltpu.VMEM((cpw, CH), jnp.int32),
            pltpu.VMEM((CH, w), jnp.float32),
            [pltpu.SemaphoreType.DMA] * NB,
        ],
    )
    def run(dst_hbm, ones_hbm, zeros_hbm, out_hbm, acc, didx, ones_v, ss):
        c = lax.axis_index("c")
        s = lax.axis_index("s")
        wid = s * NC + c
        pltpu.sync_copy(zeros_hbm, acc.at[pl.ds(s * rps, rps)])
        pltpu.sync_copy(ones_hbm, ones_v)
        pltpu.sync_copy(dst_hbm.at[wid], didx)
        plsc.subcore_barrier()

        sd = [pltpu.async_copy(ones_v, acc.at[didx.at[b]], ss[b], add=True)
              for b in range(NB)]

        @pl.loop(1, cpw // NB)
        def _(k):
            for b in range(NB):
                sd[b].wait()
                pltpu.async_copy(ones_v, acc.at[didx.at[k * NB + b]],
                                 ss[b], add=True)

        for b in range(NB):
            sd[b].wait()
        plsc.subcore_barrier()
        pltpu.sync_copy(acc.at[pl.ds(s * rps, rps)],
                        out_hbm.at[c, pl.ds(s * rps, rps)])

    return run(dst3, ones, zeros)


def _edge_pass(hn, src3, dst3, zeros, NP, cpw):
    n, h = hn.shape
    rps = NP // NS

    @functools.partial(
        pl.kernel,
        out_type=jax.ShapeDtypeStruct((NC, NP, h), jnp.float32),
        mesh=_sc_mesh(),
        scratch_types=[
            pltpu.VMEM_SHARED((NP, h), jnp.float32),
            pltpu.VMEM((cpw, CH), jnp.int32),
            [pltpu.VMEM((CH,), jnp.int32)] * 4,
            [pltpu.VMEM((CH, h), jnp.float32)] * 2,
            [pltpu.SemaphoreType.DMA] * 4,
            [pltpu.SemaphoreType.DMA] * 2,
            [pltpu.SemaphoreType.DMA] * 4,
        ],
    )
    def run(hn_hbm, src_hbm, dst_hbm, zeros_hbm, out_hbm,
            acc, didx, sidx, rows, gs, ss, isem):
        c = lax.axis_index("c")
        s = lax.axis_index("s")
        wid = s * NC + c
        pltpu.sync_copy(zeros_hbm, acc.at[pl.ds(s * rps, rps)])
        pltpu.sync_copy(dst_hbm.at[wid], didx)
        plsc.subcore_barrier()

        def idxcopy(g, q):
            return pltpu.async_copy(src_hbm.at[wid, g], sidx[q], isem[q])

        def gather(q, b):
            return pltpu.async_copy(hn_hbm.at[sidx[q]], rows[b], gs[q])

        def scatter(g, b):
            return pltpu.async_copy(rows[b], acc.at[didx.at[g]], ss[b],
                                    add=True)

        idd = [idxcopy(q, q) for q in range(4)]
        idd[0].wait()
        gd0 = gather(0, 0)
        idd[1].wait()
        gd1 = gather(1, 1)

        @pl.loop(0, cpw // 4)
        def _(k):
            g0 = k * 4
            gd0.wait()
            sda = scatter(g0, 0)
            ia0 = idxcopy(lax.rem(g0 + 4, cpw), 0)
            gd1.wait()
            sdb = scatter(g0 + 1, 1)
            ia1 = idxcopy(lax.rem(g0 + 5, cpw), 1)
            idd[2].wait()
            sda.wait()
            ga = gather(2, 0)
            idd[3].wait()
            sdb.wait()
            gb = gather(3, 1)
            ga.wait()
            sdc = scatter(g0 + 2, 0)
            idxcopy(lax.rem(g0 + 6, cpw), 2)
            gb.wait()
            sdd = scatter(g0 + 3, 1)
            idxcopy(lax.rem(g0 + 7, cpw), 3)
            ia0.wait()
            sdc.wait()
            gather(0, 0)
            ia1.wait()
            sdd.wait()
            gather(1, 1)

        gd0.wait()
        gd1.wait()
        idd[2].wait()
        idd[3].wait()
        plsc.subcore_barrier()
        pltpu.sync_copy(acc.at[pl.ds(s * rps, rps)],
                        out_hbm.at[c, pl.ds(s * rps, rps)])

    return run(hn, src3, dst3, zeros)


def _norm_from(dp):
    deg = dp[0, :, 0] + dp[1, :, 0]
    return lax.rsqrt(jnp.maximum(deg, 1.0))


def _mm(a, w):
    return jnp.dot(a, w, precision=lax.Precision.HIGHEST,
                   preferred_element_type=jnp.float32)


def _stage_a(degp, x):
    n, d = x.shape

    def body(dp_ref, x_ref, out_ref):
        norm = _norm_from(dp_ref[...])
        out_ref[...] = x_ref[...] * norm[:, None]

    return pl.pallas_call(
        body,
        grid=(n // BN,),
        in_specs=[
            pl.BlockSpec((2, BN, DW), lambda i: (0, i, 0)),
            pl.BlockSpec((BN, d), lambda i: (i, 0)),
        ],
        out_specs=pl.BlockSpec((BN, d), lambda i: (i, 0)),
        out_shape=jax.ShapeDtypeStruct((n, d), jnp.float32),
    )(degp, x)


def _stage_b(degp, aggp, W, b, n):
    d, h = W.shape

    def body(dp_ref, ap_ref, w_ref, b_ref, out_ref):
        norm = _norm_from(dp_ref[...])
        ap = ap_ref[...]
        agg = (ap[0] + ap[1]) * norm[:, None]
        hh = jnp.maximum(_mm(agg, w_ref[...]) + b_ref[...], 0.0)
        out_ref[...] = hh * norm[:, None]

    return pl.pallas_call(
        body,
        grid=(n // BN,),
        in_specs=[
            pl.BlockSpec((2, BN, DW), lambda i: (0, i, 0)),
            pl.BlockSpec((2, BN, d), lambda i: (0, i, 0)),
            pl.BlockSpec((d, h), lambda i: (0, 0)),
            pl.BlockSpec((1, h), lambda i: (0, 0)),
        ],
        out_specs=pl.BlockSpec((BN, h), lambda i: (i, 0)),
        out_shape=jax.ShapeDtypeStruct((n, h), jnp.float32),
    )(degp, aggp, W, b)


def _stage_c(degp, aggp, W2, b2, Wl1, bl1, Wl2, bl2, n):
    d, h = W2.shape
    h1 = Wl1.shape[1]
    c = Wl2.shape[1]

    def body(dp_ref, ap_ref, w2_ref, b2_ref, wl1_ref, bl1_ref,
             wl2_ref, bl2_ref, out_ref):
        norm = _norm_from(dp_ref[...])
        ap = ap_ref[...]
        agg = (ap[0] + ap[1]) * norm[:, None]
        hh = jnp.maximum(_mm(agg, w2_ref[...]) + b2_ref[...], 0.0)
        hh = jnp.maximum(_mm(hh, wl1_ref[...]) + bl1_ref[...], 0.0)
        out_ref[...] = _mm(hh, wl2_ref[...]) + bl2_ref[...]

    return pl.pallas_call(
        body,
        grid=(n // BN,),
        in_specs=[
            pl.BlockSpec((2, BN, DW), lambda i: (0, i, 0)),
            pl.BlockSpec((2, BN, d), lambda i: (0, i, 0)),
            pl.BlockSpec((d, h), lambda i: (0, 0)),
            pl.BlockSpec((1, h), lambda i: (0, 0)),
            pl.BlockSpec((h, h1), lambda i: (0, 0)),
            pl.BlockSpec((1, h1), lambda i: (0, 0)),
            pl.BlockSpec((h1, c), lambda i: (0, 0)),
            pl.BlockSpec((1, c), lambda i: (0, 0)),
        ],
        out_specs=pl.BlockSpec((BN, c), lambda i: (i, 0)),
        out_shape=jax.ShapeDtypeStruct((n, c), jnp.float32),
    )(degp, aggp, W2, b2, Wl1, bl1, Wl2, bl2)


def kernel(x, edge_index, W1, b1, W2, b2, Wl1, bl1, Wl2, bl2):
    n, d = x.shape
    h = W1.shape[1]
    e = edge_index.shape[1]

    nw = NC * NS
    cpw = NB * (-(-e // (nw * CH * NB)))
    ep = nw * CH * cpw
    np_rows = -(-(n + 1) // (NS * 8)) * (NS * 8)

    src = edge_index[0]
    dst = edge_index[1]
    pad = ep - e
    if pad:
        pi = lax.iota(jnp.int32, pad)
        src = jnp.concatenate([src, pi % n])
        dst = jnp.concatenate([dst, n + pi % (np_rows - n)])
    src3 = src.reshape(nw, cpw, CH)
    dst3 = dst.reshape(nw, cpw, CH)

    zeros_h = jnp.zeros((np_rows // NS, h), jnp.float32)
    zeros_w = jnp.zeros((np_rows // NS, DW), jnp.float32)
    ones_w = jnp.ones((CH, DW), jnp.float32)

    degp = _deg_partials(dst3, ones_w, zeros_w, np_rows, cpw)
    hn0 = _stage_a(degp, x)
    agg1 = _edge_pass(hn0, src3, dst3, zeros_h, np_rows, cpw)
    hn1 = _stage_b(degp, agg1, W1, b1.reshape(1, -1), n)
    agg2 = _edge_pass(hn1, src3, dst3, zeros_h, np_rows, cpw)
    return _stage_c(degp, agg2, W2, b2.reshape(1, -1),
                    Wl1, bl1.reshape(1, -1), Wl2, bl2.reshape(1, -1), n)
